# baseline (speedup 1.0000x reference)
import functools

import jax
import jax.numpy as jnp
from jax import lax
from jax.experimental import pallas as pl
from jax.experimental.pallas import tpu as pltpu
from jax.experimental.pallas import tpu_sc as plsc

N_NODES = 10000
NP = 10240
D = 128
NG = 64
E_BASE = 320000
E_TOT = E_BASE + N_NODES
NWORK = 32
CHUNK = 128
NCH = 82
EPT = CHUNK * NCH
E_PAD = NWORK * EPT
NEG = -1e30
F32 = jnp.float32
I32 = jnp.int32

_sc_mesh = plsc.VectorSubcoreMesh(core_axis_name="c", subcore_axis_name="s")


def _prologue_body(x_ref, w_ref, asv_ref, adv_ref, ea_ref,
                   we1_ref, ae1_ref, we2_ref, ae2_ref, we3_ref, ae3_ref,
                   h_ref, asrc_ref, adst_ref, scal_ref):
    h = jnp.dot(x_ref[...], w_ref[...], preferred_element_type=F32)
    h_ref[...] = h
    asrc_ref[...] = jnp.sum(h * asv_ref[...], axis=1, keepdims=True)
    adst_ref[...] = jnp.sum(h * adv_ref[...], axis=1, keepdims=True)
    mean = jnp.sum(ea_ref[...]) / E_BASE
    c1 = jnp.sum(we1_ref[...] * ae1_ref[...])
    c2 = jnp.sum(we2_ref[...] * ae2_ref[...])
    c3 = jnp.sum(we3_ref[...] * ae3_ref[...])
    col = lax.broadcasted_iota(I32, (8, 128), 1)
    scal_ref[...] = jnp.where(
        col == 0, mean,
        jnp.where(col == 1, c1, jnp.where(col == 2, c2,
                                          jnp.where(col == 3, c3, 0.0))))


def _prologue(x, w1, a_src1, a_dst1, ea_r, we1, ae1, we2, ae2, we3, ae3):
    return pl.pallas_call(
        _prologue_body,
        out_shape=(
            jax.ShapeDtypeStruct((NP, D), F32),
            jax.ShapeDtypeStruct((NP, 1), F32),
            jax.ShapeDtypeStruct((NP, 1), F32),
            jax.ShapeDtypeStruct((8, 128), F32),
        ),
    )(x, w1, a_src1, a_dst1, ea_r, we1, ae1, we2, ae2, we3, ae3)


def _dense_body(p_ref, sp_ref, b_ref, w_ref, asv_ref, adv_ref,
                h_ref, asrc_ref, adst_ref):
    s = sp_ref[0] + sp_ref[1]
    rs = 1.0 / (s + 1e-16)
    xp = jnp.maximum((p_ref[0] + p_ref[1]) * rs + b_ref[...], 0.0)
    h = jnp.dot(xp, w_ref[...], preferred_element_type=F32)
    h_ref[...] = h
    asrc_ref[...] = jnp.sum(h * asv_ref[...], axis=1, keepdims=True)
    adst_ref[...] = jnp.sum(h * adv_ref[...], axis=1, keepdims=True)


def _dense(outp, sp, b, w, a_src, a_dst):
    return pl.pallas_call(
        _dense_body,
        out_shape=(
            jax.ShapeDtypeStruct((NP, D), F32),
            jax.ShapeDtypeStruct((NP, 1), F32),
            jax.ShapeDtypeStruct((NP, 1), F32),
        ),
    )(outp, sp, b, w, a_src, a_dst)


def _pool_body(p_ref, sp_ref, b_ref, batch_ref, lw_ref, lb_ref, out_ref):
    s = sp_ref[0] + sp_ref[1]
    rs = 1.0 / (s + 1e-16)
    x = (p_ref[0] + p_ref[1]) * rs + b_ref[...]
    oh = (batch_ref[...] == lax.broadcasted_iota(I32, (NG, NP), 0)).astype(F32)
    sums = jnp.dot(oh, x, preferred_element_type=F32)
    cnt = jnp.sum(oh, axis=1, keepdims=True)
    pooled = sums / jnp.maximum(cnt, 1.0)
    out_ref[...] = jnp.dot(pooled, lw_ref[...],
                           preferred_element_type=F32) + lb_ref[...]


def _pool(outp, sp, b, batch_row, lin_w, lin_b):
    return pl.pallas_call(
        _pool_body,
        out_shape=jax.ShapeDtypeStruct((NG, 1), F32),
    )(outp, sp, b, batch_row, lin_w, lin_b)


@functools.partial(
    pl.kernel,
    out_type=(
        jax.ShapeDtypeStruct((NWORK, NCH, CHUNK), F32),
        jax.ShapeDtypeStruct((2, NP), F32),
    ),
    mesh=_sc_mesh,
    compiler_params=pltpu.CompilerParams(needs_layout_passes=False),
    scratch_types=[
        pltpu.VMEM((NP,), F32),
        pltpu.VMEM((NP,), F32),
        pltpu.VMEM((NCH, CHUNK), I32),
        pltpu.VMEM((NCH, CHUNK), I32),
        pltpu.VMEM((NCH, CHUNK), F32),
        pltpu.VMEM((NCH, CHUNK), F32),
        pltpu.VMEM((16,), F32),
        pltpu.VMEM((NP // 16,), F32),
        pltpu.VMEM_SHARED((NP,), F32),
        pltpu.SemaphoreType.DMA,
    ],
)
def _edge_logits(src_hbm, dst_hbm, ea_hbm, asrc_hbm, adst_hbm, c_hbm,
                 ex_hbm, sp_hbm,
                 asrc_v, adst_v, src_v, dst_v, ea_v, ex_v, c_v, z_v,
                 s_sh, sem):
    cid = lax.axis_index("c")
    sid = lax.axis_index("s")
    wid = sid * 2 + cid
    pltpu.sync_copy(asrc_hbm, asrc_v)
    pltpu.sync_copy(adst_hbm, adst_v)
    pltpu.sync_copy(c_hbm, c_v)
    pltpu.sync_copy(src_hbm.at[wid], src_v)
    pltpu.sync_copy(dst_hbm.at[wid], dst_v)
    pltpu.sync_copy(ea_hbm.at[wid], ea_v)

    stripe = NP // 16

    def zbody(i, carry):
        z_v[pl.ds(i * 16, 16)] = jnp.zeros((16,), F32)
        return carry
    lax.fori_loop(0, stripe // 16, zbody, 0)
    pltpu.sync_copy(z_v, s_sh.at[pl.ds(sid * stripe, stripe)])
    plsc.subcore_barrier()

    cvec = c_v[...]
    lane = lax.iota(I32, 16)

    def chunk_body(ch, carry):
        for g in range(CHUNK // 16):
            off = g * 16
            s16 = src_v[ch, pl.ds(off, 16)]
            d16 = dst_v[ch, pl.ds(off, 16)]
            alpha = (plsc.load_gather(asrc_v, [s16])
                     + plsc.load_gather(adst_v, [d16])
                     + cvec * ea_v[ch, pl.ds(off, 16)])
            eg = wid * EPT + ch * CHUNK + off + lane
            alpha = jnp.where(eg < E_TOT, alpha, NEG)
            alpha = jnp.where(alpha >= 0.0, alpha, 0.2 * alpha)
            ex_v[ch, pl.ds(off, 16)] = jnp.exp(alpha)
        pltpu.sync_copy(ex_v.at[ch], s_sh.at[dst_v.at[ch]], add=True)
        return carry
    lax.fori_loop(0, NCH, chunk_body, 0)

    pltpu.sync_copy(ex_v, ex_hbm.at[wid])
    plsc.subcore_barrier()

    @pl.when(sid == 0)
    def _():
        pltpu.sync_copy(s_sh, sp_hbm.at[cid])


@functools.partial(
    pl.kernel,
    out_type=jax.ShapeDtypeStruct((2, NP, D), F32),
    mesh=_sc_mesh,
    compiler_params=pltpu.CompilerParams(needs_layout_passes=False),
    scratch_types=[
        pltpu.VMEM((3, CHUNK), I32),
        pltpu.VMEM((3, CHUNK), I32),
        pltpu.VMEM((CHUNK,), I32),
        pltpu.VMEM((CHUNK,), I32),
        pltpu.VMEM((CHUNK,), F32),
        pltpu.VMEM((CHUNK,), F32),
        pltpu.VMEM((CHUNK, D), F32),
        pltpu.VMEM((CHUNK, D), F32),
        pltpu.VMEM_SHARED((NP, D), F32),
        pltpu.SemaphoreType.DMA,
        pltpu.SemaphoreType.DMA,
        pltpu.SemaphoreType.DMA,
        pltpu.SemaphoreType.DMA,
    ],
)
def _edge_aggregate(ec_hbm, h_hbm,
                    out_hbm,
                    idx0, idx1, dst0, dst1, ex0, ex1, rows0, rows1,
                    out_sh, semi0, semi1, semg0, semg1):
    cid = lax.axis_index("c")
    sid = lax.axis_index("s")
    wid = sid * 2 + cid
    LAST = NCH - 1

    idx = (idx0, idx1)
    dstb = (dst0, dst1)
    exb = (ex0, ex1)
    rows = (rows0, rows1)
    semi = (semi0, semi1)
    semg = (semg0, semg1)

    def zrow(r, carry):
        for u in range(D // 16):
            rows0[r, pl.ds(u * 16, 16)] = jnp.zeros((16,), F32)
        return carry
    lax.fori_loop(0, CHUNK, zrow, 0)
    stripe = NP // 16
    for k in range(stripe // CHUNK):
        pltpu.sync_copy(rows0,
                        out_sh.at[pl.ds(sid * stripe + k * CHUNK, CHUNK)])
    plsc.subcore_barrier()

    pltpu.sync_copy(ec_hbm.at[wid, 0], idx0)
    pltpu.async_copy(h_hbm.at[idx0.at[0]], rows0, semg0)
    pltpu.async_copy(ec_hbm.at[wid, 1], idx1, semi1)

    def sub(b, ch):
        b2 = 1 - b
        chn = jnp.minimum(ch + 1, LAST)
        pltpu.make_async_copy(h_hbm.at[idx[b].at[0]], rows[b],
                              semg[b]).wait()

        for g in range(CHUNK // 16):
            off = g * 16
            dstb[b][pl.ds(off, 16)] = idx[b][1, pl.ds(off, 16)]
            exb[b][pl.ds(off, 16)] = plsc.bitcast(idx[b][2, pl.ds(off, 16)],
                                                  F32)

        pltpu.async_copy(ec_hbm.at[wid, jnp.minimum(ch + 2, LAST)],
                         idx[b], semi[b])

        pltpu.make_async_copy(ec_hbm.at[wid, chn], idx[b2], semi[b2]).wait()
        pltpu.async_copy(h_hbm.at[idx[b2].at[0]], rows[b2], semg[b2])

        def rbody(rr, carry):
            for j in range(4):
                r = rr * 4 + j
                cb = plsc.load_gather(exb[b], [jnp.zeros((16,), I32) + r])
                for u in range(D // 16):
                    rows[b][r, pl.ds(u * 16, 16)] = \
                        rows[b][r, pl.ds(u * 16, 16)] * cb
            return carry
        lax.fori_loop(0, CHUNK // 4, rbody, 0)

        pltpu.sync_copy(rows[b], out_sh.at[dstb[b]], add=True)

    def body(i, carry):
        sub(0, 2 * i)
        sub(1, 2 * i + 1)
        return carry
    lax.fori_loop(0, NCH // 2, body, 0)

    pltpu.make_async_copy(h_hbm.at[idx0.at[0]], rows0, semg0).wait()
    pltpu.make_async_copy(ec_hbm.at[wid, LAST], idx1, semi1).wait()
    plsc.subcore_barrier()

    pltpu.sync_copy(out_sh.at[pl.ds(sid * stripe, stripe)],
                    out_hbm.at[cid, pl.ds(sid * stripe, stripe)])


def kernel(x, edge_index, edge_attr, batch,
           W1, a_src1, a_dst1, We1, a_e1, b1,
           W2, a_src2, a_dst2, We2, a_e2, b2,
           W3, a_src3, a_dst3, We3, a_e3, b3,
           lin_W, lin_b):
    f32 = F32
    x_p = jnp.zeros((NP, D), f32).at[:N_NODES].set(x.astype(f32))
    ea_r = edge_attr.astype(f32).reshape(E_BASE // D, D)

    h1, asrc1, adst1, scal = _prologue(
        x_p, W1.astype(f32),
        a_src1.reshape(1, D), a_dst1.reshape(1, D), ea_r,
        We1.reshape(1, D), a_e1.reshape(1, D),
        We2.reshape(1, D), a_e2.reshape(1, D),
        We3.reshape(1, D), a_e3.reshape(1, D))

    mean = scal[0, 0]
    cvecs = [jnp.broadcast_to(scal[0, 1], (16,)),
             jnp.broadcast_to(scal[0, 2], (16,)),
             jnp.broadcast_to(scal[0, 3], (16,))]

    loops = jnp.arange(N_NODES, dtype=I32)
    zpad = jnp.zeros((E_PAD - E_TOT,), I32)
    src3 = jnp.concatenate([edge_index[0].astype(I32), loops, zpad]
                           ).reshape(NWORK, NCH, CHUNK)
    dst3 = jnp.concatenate([edge_index[1].astype(I32), loops, zpad]
                           ).reshape(NWORK, NCH, CHUNK)
    ea3 = jnp.concatenate([
        edge_attr[:, 0].astype(f32),
        jnp.broadcast_to(mean, (N_NODES,)),
        jnp.zeros((E_PAD - E_TOT,), f32)]).reshape(NWORK, NCH, CHUNK)

    h = h1
    asrc, adst = asrc1, adst1
    wnext = [(b1, W2, a_src2, a_dst2), (b2, W3, a_src3, a_dst3)]
    outp = None
    sp = None
    for layer in range(3):
        ex3, sp = _edge_logits(src3, dst3, ea3,
                               asrc.reshape(NP), adst.reshape(NP),
                               cvecs[layer])
        ec3 = jnp.stack(
            [src3, dst3, lax.bitcast_convert_type(ex3, I32)], axis=2)
        outp = _edge_aggregate(ec3, h)
        if layer < 2:
            b_i, w_n, as_n, ad_n = wnext[layer]
            h, asrc, adst = _dense(outp, sp.reshape(2, NP, 1),
                                   b_i.reshape(1, D).astype(f32),
                                   w_n.astype(f32),
                                   as_n.reshape(1, D), ad_n.reshape(1, D))

    batch_row = jnp.full((1, NP), NG, I32).at[0, :N_NODES].set(
        batch.astype(I32))
    out = _pool(outp, sp.reshape(2, NP, 1),
                b3.reshape(1, D).astype(f32), batch_row,
                lin_W.astype(f32), lin_b.reshape(1, 1).astype(f32))
    return out[:, 0]


# speedup vs baseline: 1.1378x; 1.1378x over previous
import functools

import jax
import jax.numpy as jnp
from jax import lax
from jax.experimental import pallas as pl
from jax.experimental.pallas import tpu as pltpu
from jax.experimental.pallas import tpu_sc as plsc

N_NODES = 10000
NP = 10240
D = 128
NG = 64
E_BASE = 320000
E_TOT = E_BASE + N_NODES
NWORK = 32
CHUNK = 128
NCH = 81
EPT = CHUNK * NCH
E_PAD = NWORK * EPT
NEG = -1e30
F32 = jnp.float32
I32 = jnp.int32

_sc_mesh = plsc.VectorSubcoreMesh(core_axis_name="c", subcore_axis_name="s")


def _prologue_body(x_ref, w_ref, asv_ref, adv_ref, ea_ref,
                   we1_ref, ae1_ref, we2_ref, ae2_ref, we3_ref, ae3_ref,
                   h_ref, asrc_ref, adst_ref, scal_ref):
    h = jnp.dot(x_ref[...], w_ref[...], preferred_element_type=F32)
    h_ref[...] = h
    asrc_ref[...] = jnp.sum(h * asv_ref[...], axis=1, keepdims=True)
    adst_ref[...] = jnp.sum(h * adv_ref[...], axis=1, keepdims=True)
    mean = jnp.sum(ea_ref[...]) / E_BASE
    c1 = jnp.sum(we1_ref[...] * ae1_ref[...])
    c2 = jnp.sum(we2_ref[...] * ae2_ref[...])
    c3 = jnp.sum(we3_ref[...] * ae3_ref[...])
    col = lax.broadcasted_iota(I32, (8, 128), 1)
    scal_ref[...] = jnp.where(
        col == 0, mean,
        jnp.where(col == 1, c1, jnp.where(col == 2, c2,
                                          jnp.where(col == 3, c3, 0.0))))


def _prologue(x, w1, a_src1, a_dst1, ea_r, we1, ae1, we2, ae2, we3, ae3):
    return pl.pallas_call(
        _prologue_body,
        out_shape=(
            jax.ShapeDtypeStruct((NP, D), F32),
            jax.ShapeDtypeStruct((NP, 1), F32),
            jax.ShapeDtypeStruct((NP, 1), F32),
            jax.ShapeDtypeStruct((8, 128), F32),
        ),
    )(x, w1, a_src1, a_dst1, ea_r, we1, ae1, we2, ae2, we3, ae3)


def _dense_body(p_ref, sp_ref, b_ref, w_ref, asv_ref, adv_ref,
                h_ref, asrc_ref, adst_ref):
    s = sp_ref[0] + sp_ref[1]
    rs = 1.0 / (s + 1e-16)
    xp = jnp.maximum((p_ref[0] + p_ref[1]) * rs + b_ref[...], 0.0)
    h = jnp.dot(xp, w_ref[...], preferred_element_type=F32)
    h_ref[...] = h
    asrc_ref[...] = jnp.sum(h * asv_ref[...], axis=1, keepdims=True)
    adst_ref[...] = jnp.sum(h * adv_ref[...], axis=1, keepdims=True)


def _dense(outp, sp, b, w, a_src, a_dst):
    return pl.pallas_call(
        _dense_body,
        out_shape=(
            jax.ShapeDtypeStruct((NP, D), F32),
            jax.ShapeDtypeStruct((NP, 1), F32),
            jax.ShapeDtypeStruct((NP, 1), F32),
        ),
    )(outp, sp, b, w, a_src, a_dst)


def _pool_body(p_ref, sp_ref, b_ref, batch_ref, lw_ref, lb_ref, out_ref):
    s = sp_ref[0] + sp_ref[1]
    rs = 1.0 / (s + 1e-16)
    x = (p_ref[0] + p_ref[1]) * rs + b_ref[...]
    oh = (batch_ref[...] == lax.broadcasted_iota(I32, (NG, NP), 0)).astype(F32)
    sums = jnp.dot(oh, x, preferred_element_type=F32)
    cnt = jnp.sum(oh, axis=1, keepdims=True)
    pooled = sums / jnp.maximum(cnt, 1.0)
    out_ref[...] = jnp.dot(pooled, lw_ref[...],
                           preferred_element_type=F32) + lb_ref[...]


def _pool(outp, sp, b, batch_row, lin_w, lin_b):
    return pl.pallas_call(
        _pool_body,
        out_shape=jax.ShapeDtypeStruct((NG, 1), F32),
    )(outp, sp, b, batch_row, lin_w, lin_b)


@functools.partial(
    pl.kernel,
    out_type=(
        jax.ShapeDtypeStruct((NWORK, NCH, CHUNK), F32),
        jax.ShapeDtypeStruct((2, NP), F32),
    ),
    mesh=_sc_mesh,
    compiler_params=pltpu.CompilerParams(needs_layout_passes=False),
    scratch_types=[
        pltpu.VMEM((NP,), F32),
        pltpu.VMEM((NP,), F32),
        pltpu.VMEM((NCH, CHUNK), I32),
        pltpu.VMEM((NCH, CHUNK), I32),
        pltpu.VMEM((NCH, CHUNK), F32),
        pltpu.VMEM((NCH, CHUNK), F32),
        pltpu.VMEM((16,), F32),
        pltpu.VMEM((NP // 16,), F32),
        pltpu.VMEM_SHARED((NP,), F32),
        pltpu.SemaphoreType.DMA,
    ],
)
def _edge_logits(src_hbm, dst_hbm, ea_hbm, asrc_hbm, adst_hbm, c_hbm,
                 ex_hbm, sp_hbm,
                 asrc_v, adst_v, src_v, dst_v, ea_v, ex_v, c_v, z_v,
                 s_sh, sem):
    cid = lax.axis_index("c")
    sid = lax.axis_index("s")
    wid = sid * 2 + cid
    pltpu.sync_copy(asrc_hbm, asrc_v)
    pltpu.sync_copy(adst_hbm, adst_v)
    pltpu.sync_copy(c_hbm, c_v)
    pltpu.sync_copy(src_hbm.at[wid], src_v)
    pltpu.sync_copy(dst_hbm.at[wid], dst_v)
    pltpu.sync_copy(ea_hbm.at[wid], ea_v)

    stripe = NP // 16

    def zbody(i, carry):
        z_v[pl.ds(i * 16, 16)] = jnp.zeros((16,), F32)
        return carry
    lax.fori_loop(0, stripe // 16, zbody, 0)
    pltpu.sync_copy(z_v, s_sh.at[pl.ds(sid * stripe, stripe)])
    plsc.subcore_barrier()

    cvec = c_v[...]
    lane = lax.iota(I32, 16)

    def chunk_body(ch, carry):
        for g in range(CHUNK // 16):
            off = g * 16
            s16 = src_v[ch, pl.ds(off, 16)]
            d16 = dst_v[ch, pl.ds(off, 16)]
            alpha = (plsc.load_gather(asrc_v, [s16])
                     + plsc.load_gather(adst_v, [d16])
                     + cvec * ea_v[ch, pl.ds(off, 16)])
            eg = wid * EPT + ch * CHUNK + off + lane
            alpha = jnp.where(eg < E_TOT, alpha, NEG)
            alpha = jnp.where(alpha >= 0.0, alpha, 0.2 * alpha)
            ex_v[ch, pl.ds(off, 16)] = jnp.exp(alpha)
        pltpu.sync_copy(ex_v.at[ch], s_sh.at[dst_v.at[ch]], add=True)
        return carry
    lax.fori_loop(0, NCH, chunk_body, 0)

    pltpu.sync_copy(ex_v, ex_hbm.at[wid])
    plsc.subcore_barrier()

    @pl.when(sid == 0)
    def _():
        pltpu.sync_copy(s_sh, sp_hbm.at[cid])


@functools.partial(
    pl.kernel,
    out_type=jax.ShapeDtypeStruct((2, NP, D), F32),
    mesh=_sc_mesh,
    compiler_params=pltpu.CompilerParams(needs_layout_passes=False),
    scratch_types=[
        pltpu.VMEM((3, CHUNK), I32),
        pltpu.VMEM((CHUNK,), F32),
        pltpu.VMEM((CHUNK, D), F32),
        pltpu.VMEM_SHARED((NP, D), F32),
        pltpu.SemaphoreType.DMA,
    ],
)
def _edge_aggregate(ec_hbm, h_hbm,
                    out_hbm,
                    idx_v, coef_v, rows_v,
                    out_sh, sem):
    cid = lax.axis_index("c")
    sid = lax.axis_index("s")
    wid = sid * 2 + cid

    def zrow(r, carry):
        for u in range(D // 16):
            rows_v[r, pl.ds(u * 16, 16)] = jnp.zeros((16,), F32)
        return carry
    lax.fori_loop(0, CHUNK, zrow, 0)
    stripe = NP // 16
    for k in range(stripe // CHUNK):
        pltpu.sync_copy(rows_v, out_sh.at[pl.ds(sid * stripe + k * CHUNK, CHUNK)])
    plsc.subcore_barrier()

    def chunk_body(ch, carry):
        pltpu.sync_copy(ec_hbm.at[wid, ch], idx_v)
        pltpu.async_copy(h_hbm.at[idx_v.at[0]], rows_v, sem).wait()
        for g in range(CHUNK // 16):
            off = g * 16
            coef_v[pl.ds(off, 16)] = plsc.bitcast(idx_v[2, pl.ds(off, 16)],
                                                  F32)

        def rbody(r, c2):
            cb = plsc.load_gather(coef_v, [jnp.zeros((16,), I32) + r])
            for u in range(D // 16):
                rows_v[r, pl.ds(u * 16, 16)] = rows_v[r, pl.ds(u * 16, 16)] * cb
            return c2
        lax.fori_loop(0, CHUNK, rbody, 0)
        pltpu.sync_copy(rows_v, out_sh.at[idx_v.at[1]], add=True)
        return carry
    lax.fori_loop(0, NCH, chunk_body, 0)
    plsc.subcore_barrier()

    pltpu.sync_copy(out_sh.at[pl.ds(sid * stripe, stripe)],
                    out_hbm.at[cid, pl.ds(sid * stripe, stripe)])


def kernel(x, edge_index, edge_attr, batch,
           W1, a_src1, a_dst1, We1, a_e1, b1,
           W2, a_src2, a_dst2, We2, a_e2, b2,
           W3, a_src3, a_dst3, We3, a_e3, b3,
           lin_W, lin_b):
    f32 = F32
    x_p = jnp.zeros((NP, D), f32).at[:N_NODES].set(x.astype(f32))
    ea_r = edge_attr.astype(f32).reshape(E_BASE // D, D)

    h1, asrc1, adst1, scal = _prologue(
        x_p, W1.astype(f32),
        a_src1.reshape(1, D), a_dst1.reshape(1, D), ea_r,
        We1.reshape(1, D), a_e1.reshape(1, D),
        We2.reshape(1, D), a_e2.reshape(1, D),
        We3.reshape(1, D), a_e3.reshape(1, D))

    mean = scal[0, 0]
    cvecs = [jnp.broadcast_to(scal[0, 1], (16,)),
             jnp.broadcast_to(scal[0, 2], (16,)),
             jnp.broadcast_to(scal[0, 3], (16,))]

    loops = jnp.arange(N_NODES, dtype=I32)
    zpad = jnp.zeros((E_PAD - E_TOT,), I32)
    src3 = jnp.concatenate([edge_index[0].astype(I32), loops, zpad]
                           ).reshape(NWORK, NCH, CHUNK)
    dst3 = jnp.concatenate([edge_index[1].astype(I32), loops, zpad]
                           ).reshape(NWORK, NCH, CHUNK)
    ea3 = jnp.concatenate([
        edge_attr[:, 0].astype(f32),
        jnp.broadcast_to(mean, (N_NODES,)),
        jnp.zeros((E_PAD - E_TOT,), f32)]).reshape(NWORK, NCH, CHUNK)

    h = h1
    asrc, adst = asrc1, adst1
    wnext = [(b1, W2, a_src2, a_dst2), (b2, W3, a_src3, a_dst3)]
    outp = None
    for layer in range(3):
        ex3, sp = _edge_logits(src3, dst3, ea3,
                               asrc.reshape(NP), adst.reshape(NP),
                               cvecs[layer])
        ec3 = jnp.stack(
            [src3, dst3, lax.bitcast_convert_type(ex3, I32)], axis=2)
        outp = _edge_aggregate(ec3, h)
        if layer < 2:
            b_i, w_n, as_n, ad_n = wnext[layer]
            h, asrc, adst = _dense(outp, sp.reshape(2, NP, 1),
                                   b_i.reshape(1, D).astype(f32),
                                   w_n.astype(f32),
                                   as_n.reshape(1, D), ad_n.reshape(1, D))

    batch_row = jnp.full((1, NP), NG, I32).at[0, :N_NODES].set(
        batch.astype(I32))
    out = _pool(outp, sp.reshape(2, NP, 1),
                b3.reshape(1, D).astype(f32), batch_row,
                lin_W.astype(f32), lin_b.reshape(1, 1).astype(f32))
    return out[:, 0]
